# baseline (speedup 1.0000x reference)
import jax
import jax.numpy as jnp
from jax.experimental import pallas as pl
from jax.experimental.pallas import tpu as pltpu

N_TOKENS = 16384
IN_FEATURES = 4096
N_EXPERTS = 64
TOP_K = 8
CH = 512
NBUF = 4
NCHUNKS = N_TOKENS // CH


def _compute_chunk(xc, wt_ref, w_out_ref, i_out_ref, row0):
    logits = jnp.dot(xc, wt_ref[...], preferred_element_type=jnp.float32)
    iota = jax.lax.broadcasted_iota(jnp.int32, (CH, N_EXPERTS), 1)

    l = logits
    tops = []
    idxs = []
    for j in range(TOP_K):
        cur = jnp.max(l, axis=1, keepdims=True)
        hit = l == cur
        idx = jnp.min(jnp.where(hit, iota, N_EXPERTS), axis=1, keepdims=True)
        tops.append(cur)
        idxs.append(idx)
        l = jnp.where(hit, float("-inf"), l)

    m = tops[0]
    z = jnp.sum(jnp.exp(logits - m), axis=1, keepdims=True)
    w = jnp.concatenate([jnp.exp(t - m) / z for t in tops], axis=1)
    i = jnp.concatenate(idxs, axis=1)
    w_out_ref[pl.ds(row0, CH), :] = w
    i_out_ref[pl.ds(row0, CH), :] = i


def _gate_kernel(x_hbm, wt_ref, w_out_ref, i_out_ref, xbuf, sems):
    def start(c, slot):
        pltpu.make_async_copy(
            x_hbm.at[pl.ds(c * CH, CH), :], xbuf.at[slot], sems.at[slot]
        ).start()

    for s in range(NBUF):
        start(s, s)

    def body(c, _):
        slot = jax.lax.rem(c, NBUF)
        pltpu.make_async_copy(
            x_hbm.at[pl.ds(c * CH, CH), :], xbuf.at[slot], sems.at[slot]
        ).wait()
        _compute_chunk(xbuf[slot], wt_ref, w_out_ref, i_out_ref, c * CH)

        @pl.when(c + NBUF < NCHUNKS)
        def _():
            start(c + NBUF, slot)

        return ()

    jax.lax.fori_loop(0, NCHUNKS, body, (), unroll=False)


def kernel(x, W):
    wt = W.T
    weights, indices = pl.pallas_call(
        _gate_kernel,
        in_specs=[
            pl.BlockSpec(memory_space=pl.ANY),
            pl.BlockSpec(memory_space=pltpu.VMEM),
        ],
        out_specs=[
            pl.BlockSpec(memory_space=pltpu.VMEM),
            pl.BlockSpec(memory_space=pltpu.VMEM),
        ],
        out_shape=[
            jax.ShapeDtypeStruct((N_TOKENS, TOP_K), jnp.float32),
            jax.ShapeDtypeStruct((N_TOKENS, TOP_K), jnp.int32),
        ],
        scratch_shapes=[
            pltpu.VMEM((NBUF, CH, IN_FEATURES), jnp.float32),
            pltpu.SemaphoreType.DMA((NBUF,)),
        ],
    )(x, wt)
    return weights, indices


# speedup vs baseline: 1.4977x; 1.4977x over previous
import jax
import jax.numpy as jnp
from jax.experimental import pallas as pl

N_TOKENS = 16384
IN_FEATURES = 4096
N_EXPERTS = 64
TOP_K = 8
BM = 1024


def _gate_kernel(x_ref, w_ref, w_out_ref, i_out_ref):
    lt = jax.lax.dot_general(
        w_ref[...],
        x_ref[...],
        (((1,), (1,)), ((), ())),
        preferred_element_type=jnp.float32,
    )
    iota = jax.lax.broadcasted_iota(jnp.int32, (N_EXPERTS, BM), 0)

    l = lt
    tops = []
    idxs = []
    for j in range(TOP_K):
        cur = jnp.max(l, axis=0, keepdims=True)
        hit = l == cur
        idx = jnp.min(jnp.where(hit, iota, N_EXPERTS), axis=0, keepdims=True)
        tops.append(cur)
        idxs.append(idx)
        l = jnp.where(hit, float("-inf"), l)

    m = tops[0]
    z = jnp.sum(jnp.exp(lt - m), axis=0, keepdims=True)
    for j in range(TOP_K):
        w_out_ref[j : j + 1, :] = jnp.exp(tops[j] - m) / z
        i_out_ref[j : j + 1, :] = idxs[j]


def kernel(x, W):
    grid = (N_TOKENS // BM,)
    weights_t, indices_t = pl.pallas_call(
        _gate_kernel,
        grid=grid,
        in_specs=[
            pl.BlockSpec((BM, IN_FEATURES), lambda i: (i, 0)),
            pl.BlockSpec((N_EXPERTS, IN_FEATURES), lambda i: (0, 0)),
        ],
        out_specs=[
            pl.BlockSpec((TOP_K, BM), lambda i: (0, i)),
            pl.BlockSpec((TOP_K, BM), lambda i: (0, i)),
        ],
        out_shape=[
            jax.ShapeDtypeStruct((TOP_K, N_TOKENS), jnp.float32),
            jax.ShapeDtypeStruct((TOP_K, N_TOKENS), jnp.int32),
        ],
    )(x, W)
    return weights_t.T, indices_t.T
